# baseline (speedup 1.0000x reference)
import functools

import jax
import jax.numpy as jnp
from jax import lax
from jax.experimental import pallas as pl
from jax.experimental.pallas import tpu as pltpu
from jax.experimental.pallas import tpu_sc as plsc

DIM = 1024
INTER = 512
SHARED_INTER = 2048
NUM_EXPERTS = 64
T = 32
N_SHARED_CHUNKS = SHARED_INTER // INTER
NBUF = 4
LOOKAHEAD = 3
VLEN = 2 * NUM_EXPERTS
GCHUNK = 256


def _vmax16(v):
    m = v[0]
    for j in range(1, 16):
        m = jnp.maximum(m, v[j])
    return m


def _vmin16(v):
    m = v[0]
    for j in range(1, 16):
        m = jnp.minimum(m, v[j])
    return m


def _sc_router(gate_t_hbm, x_hbm, coef_hbm, gbuf, xbuf, cbuf):
    cid = lax.axis_index("c")
    sid = lax.axis_index("s")
    wid = sid * 2 + cid
    pltpu.sync_copy(x_hbm.at[wid], xbuf)

    z = jnp.zeros((16,), jnp.float32)
    scores = (z, z, z, z)
    for r in range(DIM // GCHUNK):
        pltpu.sync_copy(gate_t_hbm.at[pl.ds(r * GCHUNK, GCHUNK)], gbuf)

        def k_step(kk, accs, r=r):
            xv = xbuf[pl.ds(r * GCHUNK + 16 * kk, 16)]
            a = list(accs)
            for j in range(16):
                xj = xv[j]
                row = 16 * kk + j
                for c in range(4):
                    a[c] = a[c] + xj * gbuf[row, pl.ds(16 * c, 16)]
            return tuple(a)

        scores = lax.fori_loop(0, GCHUNK // 16, k_step, scores)

    idx = [lax.iota(jnp.int32, 16) + 16 * c for c in range(4)]
    m1 = _vmax16(jnp.maximum(jnp.maximum(scores[0], scores[1]),
                             jnp.maximum(scores[2], scores[3])))
    a1 = _vmin16(jnp.minimum(
        jnp.minimum(jnp.where(scores[0] == m1, idx[0], NUM_EXPERTS),
                    jnp.where(scores[1] == m1, idx[1], NUM_EXPERTS)),
        jnp.minimum(jnp.where(scores[2] == m1, idx[2], NUM_EXPERTS),
                    jnp.where(scores[3] == m1, idx[3], NUM_EXPERTS))))
    masked = [jnp.where(idx[c] == a1, jnp.float32(-jnp.inf), scores[c])
              for c in range(4)]
    m2 = _vmax16(jnp.maximum(jnp.maximum(masked[0], masked[1]),
                             jnp.maximum(masked[2], masked[3])))
    a2 = _vmin16(jnp.minimum(
        jnp.minimum(jnp.where(masked[0] == m2, idx[0], NUM_EXPERTS),
                    jnp.where(masked[1] == m2, idx[1], NUM_EXPERTS)),
        jnp.minimum(jnp.where(masked[2] == m2, idx[2], NUM_EXPERTS),
                    jnp.where(masked[3] == m2, idx[3], NUM_EXPERTS))))
    cbuf[pl.ds(0, 16)] = z + (m2 - m1)
    e2v = jnp.exp(cbuf[pl.ds(0, 16)])
    denom = 1.0 + e2v
    s1v = (1.0 + z) / denom
    s2v = e2v / denom
    for c in range(4):
        cbuf[pl.ds(16 * c, 16)] = (jnp.where(idx[c] == a1, s1v, z)
                                   + jnp.where(idx[c] == a2, s2v, z))
    pltpu.sync_copy(cbuf, coef_hbm.at[wid])


def _shared_body(x_ref, sg_ref, su_ref, sd_ref, out_ref):
    i = pl.program_id(0)

    @pl.when(i == 0)
    def _init():
        out_ref[...] = jnp.zeros_like(out_ref)

    xb = x_ref[...].astype(jnp.bfloat16)
    hg = jnp.dot(xb, sg_ref[...].astype(jnp.bfloat16).T,
                 preferred_element_type=jnp.float32)
    hu = jnp.dot(xb, su_ref[...].astype(jnp.bfloat16).T,
                 preferred_element_type=jnp.float32)
    h = (hg * jax.lax.logistic(hg) * hu).astype(jnp.bfloat16)
    out_ref[...] += jax.lax.dot_general(
        h, sd_ref[...].astype(jnp.bfloat16), (((1,), (1,)), ((), ())),
        preferred_element_type=jnp.float32)


def _expert_body(x_ref, coef_ref, shared_ref, w1_hbm, w3_hbm, w2_hbm,
                 out_ref, xb_ref, visv_ref, viss_ref, w1b, w3b, w2b,
                 sems, sem_vs):
    def issue(j, slot):
        e = viss_ref[0, j]
        pltpu.make_async_copy(w1_hbm.at[e], w1b.at[slot],
                              sems.at[slot, 0]).start()
        pltpu.make_async_copy(w3_hbm.at[e], w3b.at[slot],
                              sems.at[slot, 1]).start()
        pltpu.make_async_copy(w2_hbm.at[e], w2b.at[slot],
                              sems.at[slot, 2]).start()

    coef = coef_ref[...]
    out_ref[...] = shared_ref[...]
    xb_ref[...] = x_ref[...].astype(jnp.bfloat16)

    act_row = (jnp.max(coef, axis=0, keepdims=True) > 0.0
               ).astype(jnp.float32)
    r64 = jax.lax.broadcasted_iota(jnp.int32, (NUM_EXPERTS, NUM_EXPERTS), 0)
    c64 = jax.lax.broadcasted_iota(jnp.int32, (NUM_EXPERTS, NUM_EXPERTS), 1)
    ident = (r64 == c64).astype(jnp.float32)
    act_col = jax.lax.dot_general(
        ident, act_row, (((1,), (1,)), ((), ())),
        preferred_element_type=jnp.float32)
    j_ge_e = (r64 >= c64).astype(jnp.float32)
    pos_col = jnp.dot(j_ge_e, act_col, preferred_element_type=jnp.float32)
    n_active = jnp.max(pos_col)
    rw = jax.lax.broadcasted_iota(jnp.int32, (NUM_EXPERTS, VLEN), 0)
    cw = jax.lax.broadcasted_iota(jnp.int32, (NUM_EXPERTS, VLEN), 1)
    slot_hit = (pos_col - 1.0) == cw.astype(jnp.float32)
    visit_raw = jnp.sum(rw.astype(jnp.float32) * act_col * slot_hit,
                        axis=0, keepdims=True)
    e_col = jax.lax.broadcasted_iota(
        jnp.int32, (NUM_EXPERTS, 1), 0).astype(jnp.float32)
    last_active = jnp.max(e_col * act_col)
    j_row = jax.lax.broadcasted_iota(jnp.int32, (1, VLEN), 1)
    vis = jnp.where(j_row.astype(jnp.float32) < n_active, visit_raw,
                    last_active)
    vis = jnp.where(j_row == NUM_EXPERTS, n_active, vis)
    visv_ref[...] = vis.astype(jnp.int32)
    cp = pltpu.make_async_copy(visv_ref, viss_ref, sem_vs)
    cp.start()
    cp.wait()
    issue(0, 0)
    issue(1, 1)
    n_act = viss_ref[0, NUM_EXPERTS]

    @pl.when(n_act > 2)
    def _issue2():
        issue(2, 2)

    xb = xb_ref[...]

    def loop(j, carry):
        slot = jax.lax.rem(j, NBUF)
        e = viss_ref[0, j]
        pltpu.make_async_copy(w1_hbm.at[e], w1b.at[slot],
                              sems.at[slot, 0]).wait()

        @pl.when(j + LOOKAHEAD < n_act)
        def _prefetch():
            issue(j + LOOKAHEAD, jax.lax.rem(j + LOOKAHEAD, NBUF))

        h1 = jnp.dot(xb, w1b[slot].astype(jnp.bfloat16).T,
                     preferred_element_type=jnp.float32)
        pltpu.make_async_copy(w3_hbm.at[e], w3b.at[slot],
                              sems.at[slot, 1]).wait()
        h3 = jnp.dot(xb, w3b[slot].astype(jnp.bfloat16).T,
                     preferred_element_type=jnp.float32)
        pltpu.make_async_copy(w2_hbm.at[e], w2b.at[slot],
                              sems.at[slot, 2]).wait()
        g = h1 * jax.lax.logistic(h1) * h3
        e_ids = jax.lax.broadcasted_iota(jnp.int32, (T, NUM_EXPERTS), 1)
        c = jnp.sum(jnp.where(e_ids == e, coef_ref[...], 0.0), axis=1,
                    keepdims=True)
        out_ref[...] += jnp.dot((g * c).astype(jnp.bfloat16),
                                w2b[slot].astype(jnp.bfloat16),
                                preferred_element_type=jnp.float32)
        return carry

    jax.lax.fori_loop(0, n_act, loop, 0)


@jax.jit
def kernel(x, gate_w, w1, w2, w3, shared_gate_w, shared_up_w, shared_down_w):
    orig_shape = x.shape
    x_flat = x.reshape(-1, DIM)
    gate_t = gate_w.T

    mesh = plsc.VectorSubcoreMesh(core_axis_name="c", subcore_axis_name="s")
    coef = pl.kernel(
        _sc_router,
        out_type=jax.ShapeDtypeStruct((T, NUM_EXPERTS), jnp.float32),
        mesh=mesh,
        scratch_types=[
            pltpu.VMEM((GCHUNK, NUM_EXPERTS), jnp.float32),
            pltpu.VMEM((DIM,), jnp.float32),
            pltpu.VMEM((NUM_EXPERTS,), jnp.float32),
        ],
    )(gate_t, x_flat)

    shared_out = pl.pallas_call(
        _shared_body,
        grid=(N_SHARED_CHUNKS,),
        in_specs=[
            pl.BlockSpec((T, DIM), lambda i: (0, 0)),
            pl.BlockSpec((INTER, DIM), lambda i: (i, 0)),
            pl.BlockSpec((INTER, DIM), lambda i: (i, 0)),
            pl.BlockSpec((DIM, INTER), lambda i: (0, i)),
        ],
        out_specs=pl.BlockSpec((T, DIM), lambda i: (0, 0)),
        out_shape=jax.ShapeDtypeStruct((T, DIM), jnp.float32),
        compiler_params=pltpu.CompilerParams(
            dimension_semantics=("arbitrary",)),
    )(x_flat, shared_gate_w, shared_up_w, shared_down_w)

    out = pl.pallas_call(
        _expert_body,
        in_specs=[
            pl.BlockSpec((T, DIM), lambda: (0, 0)),
            pl.BlockSpec((T, NUM_EXPERTS), lambda: (0, 0)),
            pl.BlockSpec((T, DIM), lambda: (0, 0)),
            pl.BlockSpec(memory_space=pl.ANY),
            pl.BlockSpec(memory_space=pl.ANY),
            pl.BlockSpec(memory_space=pl.ANY),
        ],
        out_specs=pl.BlockSpec((T, DIM), lambda: (0, 0)),
        out_shape=jax.ShapeDtypeStruct((T, DIM), jnp.float32),
        scratch_shapes=[
            pltpu.VMEM((T, DIM), jnp.bfloat16),
            pltpu.VMEM((1, VLEN), jnp.int32),
            pltpu.SMEM((1, VLEN), jnp.int32),
            pltpu.VMEM((NBUF, INTER, DIM), jnp.float32),
            pltpu.VMEM((NBUF, INTER, DIM), jnp.float32),
            pltpu.VMEM((NBUF, INTER, DIM), jnp.float32),
            pltpu.SemaphoreType.DMA((NBUF, 3)),
            pltpu.SemaphoreType.DMA,
        ],
    )(x_flat, coef, shared_out, w1, w3, w2)

    return out.reshape(orig_shape)


# speedup vs baseline: 1.3835x; 1.3835x over previous
import jax
import jax.numpy as jnp
from jax.experimental import pallas as pl
from jax.experimental.pallas import tpu as pltpu

DIM = 1024
INTER = 512
SHARED_INTER = 2048
NUM_EXPERTS = 64
T = 32
NBUF = 4
LOOKAHEAD = 3
VLEN = 2 * NUM_EXPERTS


def _body(x_ref, gate_ref, w1_hbm, w3_hbm, w2_hbm, sg_hbm, su_hbm, sd_hbm,
          out_ref, visv_ref, viss_ref, sgb, sub, sdb, w1b, w3b, w2b,
          sems, ssems, sem_vs):
    cp_sg = pltpu.make_async_copy(sg_hbm, sgb, ssems.at[0])
    cp_su = pltpu.make_async_copy(su_hbm, sub, ssems.at[1])
    cp_sd = pltpu.make_async_copy(sd_hbm, sdb, ssems.at[2])
    cp_sg.start()
    cp_su.start()
    cp_sd.start()

    def issue(j, slot):
        e = viss_ref[0, j]
        pltpu.make_async_copy(w1_hbm.at[e], w1b.at[slot],
                              sems.at[slot, 0]).start()
        pltpu.make_async_copy(w3_hbm.at[e], w3b.at[slot],
                              sems.at[slot, 1]).start()
        pltpu.make_async_copy(w2_hbm.at[e], w2b.at[slot],
                              sems.at[slot, 2]).start()

    xv = x_ref[...]
    xb = xv.astype(jnp.bfloat16)
    scores = jnp.dot(xv, gate_ref[...].T,
                     preferred_element_type=jnp.float32)
    e_ids = jax.lax.broadcasted_iota(jnp.int32, (T, NUM_EXPERTS), 1)
    m1 = jnp.max(scores, axis=1, keepdims=True)
    a1 = jnp.min(jnp.where(scores == m1, e_ids, NUM_EXPERTS), axis=1,
                 keepdims=True)
    masked = jnp.where(e_ids == a1, -jnp.inf, scores)
    m2 = jnp.max(masked, axis=1, keepdims=True)
    a2 = jnp.min(jnp.where(masked == m2, e_ids, NUM_EXPERTS), axis=1,
                 keepdims=True)
    e2 = jnp.exp(m2 - m1)
    s1 = 1.0 / (1.0 + e2)
    s2 = e2 / (1.0 + e2)
    coef = (jnp.where(e_ids == a1, s1, 0.0)
            + jnp.where(e_ids == a2, s2, 0.0))

    act_row = (jnp.max(coef, axis=0, keepdims=True) > 0.0
               ).astype(jnp.float32)
    r64 = jax.lax.broadcasted_iota(jnp.int32, (NUM_EXPERTS, NUM_EXPERTS), 0)
    c64 = jax.lax.broadcasted_iota(jnp.int32, (NUM_EXPERTS, NUM_EXPERTS), 1)
    ident = (r64 == c64).astype(jnp.float32)
    act_col = jax.lax.dot_general(
        ident, act_row, (((1,), (1,)), ((), ())),
        preferred_element_type=jnp.float32)
    j_ge_e = (r64 >= c64).astype(jnp.float32)
    pos_col = jnp.dot(j_ge_e, act_col,
                      preferred_element_type=jnp.float32)
    n_active = jnp.max(pos_col)
    rw = jax.lax.broadcasted_iota(jnp.int32, (NUM_EXPERTS, VLEN), 0)
    cw = jax.lax.broadcasted_iota(jnp.int32, (NUM_EXPERTS, VLEN), 1)
    slot_hit = (pos_col - 1.0) == cw.astype(jnp.float32)
    visit_raw = jnp.sum(rw.astype(jnp.float32) * act_col * slot_hit,
                        axis=0, keepdims=True)
    e_col = jax.lax.broadcasted_iota(
        jnp.int32, (NUM_EXPERTS, 1), 0).astype(jnp.float32)
    last_active = jnp.max(e_col * act_col)
    j_row = jax.lax.broadcasted_iota(jnp.int32, (1, VLEN), 1)
    vis = jnp.where(j_row.astype(jnp.float32) < n_active, visit_raw,
                    last_active)
    vis = jnp.where(j_row == NUM_EXPERTS, n_active, vis)
    visv_ref[...] = vis.astype(jnp.int32)
    cp = pltpu.make_async_copy(visv_ref, viss_ref, sem_vs)
    cp.start()
    cp.wait()
    issue(0, 0)
    issue(1, 1)
    n_act = viss_ref[0, NUM_EXPERTS]

    @pl.when(n_act > 2)
    def _issue2():
        issue(2, 2)

    cp_sg.wait()
    hg = jnp.dot(xb, sgb[...].astype(jnp.bfloat16).T,
                 preferred_element_type=jnp.float32)
    cp_su.wait()
    hu = jnp.dot(xb, sub[...].astype(jnp.bfloat16).T,
                 preferred_element_type=jnp.float32)
    h = (hg * jax.lax.logistic(hg) * hu).astype(jnp.bfloat16)
    cp_sd.wait()
    out_ref[...] = jax.lax.dot_general(
        h, sdb[...].astype(jnp.bfloat16), (((1,), (1,)), ((), ())),
        preferred_element_type=jnp.float32)

    def loop(j, carry):
        slot = jax.lax.rem(j, NBUF)
        e = viss_ref[0, j]
        pltpu.make_async_copy(w1_hbm.at[e], w1b.at[slot],
                              sems.at[slot, 0]).wait()

        @pl.when(j + LOOKAHEAD < n_act)
        def _prefetch():
            issue(j + LOOKAHEAD, jax.lax.rem(j + LOOKAHEAD, NBUF))

        h1 = jnp.dot(xb, w1b[slot].astype(jnp.bfloat16).T,
                     preferred_element_type=jnp.float32)
        pltpu.make_async_copy(w3_hbm.at[e], w3b.at[slot],
                              sems.at[slot, 1]).wait()
        h3 = jnp.dot(xb, w3b[slot].astype(jnp.bfloat16).T,
                     preferred_element_type=jnp.float32)
        pltpu.make_async_copy(w2_hbm.at[e], w2b.at[slot],
                              sems.at[slot, 2]).wait()
        g = h1 * jax.lax.logistic(h1) * h3
        c = jnp.sum(jnp.where(e_ids == e, coef, 0.0), axis=1,
                    keepdims=True)
        out_ref[...] += jnp.dot((g * c).astype(jnp.bfloat16),
                                w2b[slot].astype(jnp.bfloat16),
                                preferred_element_type=jnp.float32)
        return carry

    jax.lax.fori_loop(0, n_act, loop, 0)


@jax.jit
def kernel(x, gate_w, w1, w2, w3, shared_gate_w, shared_up_w, shared_down_w):
    orig_shape = x.shape
    x_flat = x.reshape(-1, DIM)

    out = pl.pallas_call(
        _body,
        in_specs=[
            pl.BlockSpec((T, DIM), lambda: (0, 0)),
            pl.BlockSpec((NUM_EXPERTS, DIM), lambda: (0, 0)),
            pl.BlockSpec(memory_space=pl.ANY),
            pl.BlockSpec(memory_space=pl.ANY),
            pl.BlockSpec(memory_space=pl.ANY),
            pl.BlockSpec(memory_space=pl.ANY),
            pl.BlockSpec(memory_space=pl.ANY),
            pl.BlockSpec(memory_space=pl.ANY),
        ],
        out_specs=pl.BlockSpec((T, DIM), lambda: (0, 0)),
        out_shape=jax.ShapeDtypeStruct((T, DIM), jnp.float32),
        scratch_shapes=[
            pltpu.VMEM((1, VLEN), jnp.int32),
            pltpu.SMEM((1, VLEN), jnp.int32),
            pltpu.VMEM((SHARED_INTER, DIM), jnp.float32),
            pltpu.VMEM((SHARED_INTER, DIM), jnp.float32),
            pltpu.VMEM((DIM, SHARED_INTER), jnp.float32),
            pltpu.VMEM((NBUF, INTER, DIM), jnp.float32),
            pltpu.VMEM((NBUF, INTER, DIM), jnp.float32),
            pltpu.VMEM((NBUF, INTER, DIM), jnp.float32),
            pltpu.SemaphoreType.DMA((NBUF, 3)),
            pltpu.SemaphoreType.DMA((3,)),
            pltpu.SemaphoreType.DMA,
        ],
    )(x_flat, gate_w, w1, w3, w2, shared_gate_w, shared_up_w, shared_down_w)

    return out.reshape(orig_shape)
